# baseline (speedup 1.0000x reference)
import functools

import jax
import jax.numpy as jnp
from jax import lax
from jax.experimental import pallas as pl
from jax.experimental.pallas import tpu as pltpu
from jax.experimental.pallas import tpu_sc as plsc

_B = 16384
_C = 1000
_ROWS = 512
_NEG = -3.0e38


_NC = 2
_NS = 16
_NW = _NC * _NS
_RPW = _B // _NW
_CH = 32
_NCHUNK = _RPW // _CH


def _sc_margin_body(o_hbm, t_hbm, out_hbm, tgt_v, buf0, buf1, marg_v,
                    sem0, sem1):
    wid = lax.axis_index("s") * _NC + lax.axis_index("c")
    base = wid * _RPW
    pltpu.sync_copy(t_hbm.at[pl.ds(base, _RPW)], tgt_v)

    lane = lax.iota(jnp.int32, 16)
    neg = jnp.full((16,), _NEG, jnp.float32)
    bufs = (buf0, buf1)
    sems = (sem0, sem1)

    def process_chunk(buf, chunk, carry0):
        for gg in range(_CH // 16):

            def row_body(rr, carry):
                m1p, m2p = carry
                row = gg * 16 + rr

                def step(i, c2):
                    m1v, m2v = c2
                    v = buf[row, pl.ds(i * 16, 16)]
                    m2v = jnp.maximum(m2v, jnp.minimum(m1v, v))
                    m1v = jnp.maximum(m1v, v)
                    return (m1v, m2v)

                m1v, m2v = lax.fori_loop(0, 62, step, (neg, neg))
                tv = buf[row, pl.ds(984, 16)]
                tv = jnp.where(lane < 8, _NEG, tv)
                m2v = jnp.maximum(m2v, jnp.minimum(m1v, tv))
                m1v = jnp.maximum(m1v, tv)
                M1 = jnp.max(m1v)
                eq = m1v == M1
                cnt = plsc.all_reduce_population_count(eq)
                r2 = jnp.maximum(jnp.max(jnp.where(eq, _NEG, m1v)),
                                 jnp.max(m2v))
                M2 = jnp.where(cnt > 1, M1, r2)
                m1p = jnp.where(lane == rr, M1, m1p)
                m2p = jnp.where(lane == rr, M2, m2p)
                return (m1p, m2p)

            m1p, m2p = lax.fori_loop(0, 16, row_body, (neg, neg))
            tgt16 = tgt_v[pl.ds(chunk * _CH + gg * 16, 16)]
            rows = gg * 16 + lane
            tlv = plsc.load_gather(buf, [rows, tgt16])
            margv = jnp.where(m1p == tlv, m1p - m2p, jnp.float32(0.0))
            marg_v[pl.ds(chunk * _CH + gg * 16, 16)] = margv
        return carry0

    def chunk_slice(g):
        return o_hbm.at[pl.ds(base + g * _CH, _CH), :]

    cp0 = pltpu.async_copy(chunk_slice(0), buf0, sem0)

    def pair_body(p, carry):
        cp1 = pltpu.async_copy(chunk_slice(2 * p + 1), buf1, sem1)
        pltpu.make_async_copy(chunk_slice(2 * p), buf0, sem0).wait()
        carry = process_chunk(buf0, 2 * p, carry)
        cp2 = pltpu.async_copy(chunk_slice(2 * p + 2), buf0, sem0)
        pltpu.make_async_copy(chunk_slice(2 * p + 1), buf1, sem1).wait()
        carry = process_chunk(buf1, 2 * p + 1, carry)
        return carry

    carry = lax.fori_loop(0, _NCHUNK // 2 - 1, pair_body, 0)
    cp1 = pltpu.async_copy(chunk_slice(_NCHUNK - 1), buf1, sem1)
    pltpu.make_async_copy(chunk_slice(_NCHUNK - 2), buf0, sem0).wait()
    carry = process_chunk(buf0, _NCHUNK - 2, carry)
    pltpu.make_async_copy(chunk_slice(_NCHUNK - 1), buf1, sem1).wait()
    carry = process_chunk(buf1, _NCHUNK - 1, carry)
    pltpu.sync_copy(marg_v, out_hbm.at[pl.ds(base, _RPW)])


@jax.jit
def _sc_margins(o_flat, targets):
    mesh = plsc.VectorSubcoreMesh(core_axis_name="c", subcore_axis_name="s")
    return pl.kernel(
        _sc_margin_body,
        mesh=mesh,
        out_type=jax.ShapeDtypeStruct((_B,), jnp.float32),
        scratch_types=[
            pltpu.VMEM((_RPW,), jnp.int32),
            pltpu.VMEM((_CH, _C), jnp.float32),
            pltpu.VMEM((_CH, _C), jnp.float32),
            pltpu.VMEM((_RPW,), jnp.float32),
            pltpu.SemaphoreType.DMA,
            pltpu.SemaphoreType.DMA,
        ],
        compiler_params=pltpu.CompilerParams(needs_layout_passes=False),
    )(o_flat, targets)


def _tc_body(o1, o2, o3, o4, tgt, out, mx):
    t = tgt[:, 0]
    col = jax.lax.broadcasted_iota(jnp.int32, (_ROWS, _C), 1)
    tmask = col == t[:, None]

    def margin(o):
        m1 = jnp.max(o, axis=1)
        tl = jnp.sum(jnp.where(tmask, o, jnp.float32(0.0)), axis=1)
        mx2 = jnp.max(jnp.where(tmask, _NEG, o), axis=1)
        return jnp.where(m1 == tl, m1 - mx2, jnp.float32(0.0)), m1

    d1, x1 = margin(o1[...])
    d2, x2 = margin(o2[...])
    d3, x3 = margin(o3[...])
    d4, x4 = margin(o4[...])
    out[...] = jnp.stack([d1, d2, d3, d4], axis=1)

    bmax = jnp.max(jnp.maximum(jnp.maximum(x1, x2), jnp.maximum(x3, x4)))

    @pl.when(pl.program_id(0) == 0)
    def _():
        mx[...] = bmax[None, None]

    @pl.when(pl.program_id(0) != 0)
    def _():
        mx[...] = jnp.maximum(mx[...], bmax[None, None])


def _combine_body(d14, d5, out):
    preds = jnp.concatenate([d14[...], d5[...]], axis=1) * jnp.float32(0.5)
    preds = preds - jnp.max(preds, axis=1, keepdims=True)
    e = jnp.exp(preds)
    out[...] = e / jnp.sum(e, axis=1, keepdims=True)


@jax.jit
def _run(o1, o2, o3, o4, o5, targets):
    d5 = _sc_margins(o5, targets)

    grid = (_B // _ROWS,)
    ospec = pl.BlockSpec((_ROWS, _C), lambda i: (i, 0))
    d14, mx = pl.pallas_call(
        _tc_body,
        grid=grid,
        in_specs=[ospec, ospec, ospec, ospec,
                  pl.BlockSpec((_ROWS, 1), lambda i: (i, 0))],
        out_specs=[pl.BlockSpec((_ROWS, 4), lambda i: (i, 0)),
                   pl.BlockSpec((1, 1), lambda i: (0, 0))],
        out_shape=[jax.ShapeDtypeStruct((_B, 4), jnp.float32),
                   jax.ShapeDtypeStruct((1, 1), jnp.float32)],
        compiler_params=pltpu.CompilerParams(
            dimension_semantics=("arbitrary",)),
    )(o1, o2, o3, o4, targets.reshape(_B, 1))

    out = pl.pallas_call(
        _combine_body,
        out_shape=jax.ShapeDtypeStruct((_B, 5), jnp.float32),
    )(d14, d5.reshape(_B, 1))
    return mx[0, 0], out


def kernel(outputs1, outputs2, outputs3, outputs4, mimic, targets, n_test):
    mx, out = _run(outputs1, outputs2, outputs3, outputs4, mimic, targets)
    return mx, out


# speedup vs baseline: 1.0042x; 1.0042x over previous
import functools

import jax
import jax.numpy as jnp
from jax import lax
from jax.experimental import pallas as pl
from jax.experimental.pallas import tpu as pltpu
from jax.experimental.pallas import tpu_sc as plsc

_B = 16384
_C = 1000
_ROWS = 512
_NEG = -3.0e38


_NC = 2
_NS = 16
_NW = _NC * _NS
_RPW = _B // _NW
_CH = 32
_NCHUNK = _RPW // _CH


def _sc_margin_body(o_hbm, t_hbm, out_hbm, tgt_v, buf0, buf1, marg_v,
                    sem0, sem1):
    wid = lax.axis_index("s") * _NC + lax.axis_index("c")
    base = wid * _RPW
    pltpu.sync_copy(t_hbm.at[pl.ds(base, _RPW)], tgt_v)

    lane = lax.iota(jnp.int32, 16)
    neg = jnp.full((16,), _NEG, jnp.float32)
    bufs = (buf0, buf1)
    sems = (sem0, sem1)

    def process_chunk(buf, chunk, carry0):
        for gg in range(_CH // 16):

            def row_body(rr, carry):
                m1p, m2p = carry
                row = gg * 16 + rr

                acc = [[neg, neg], [neg, neg]]
                for i in range(62):
                    v = buf[row, pl.ds(i * 16, 16)]
                    a = acc[i % 2]
                    a[1] = jnp.maximum(a[1], jnp.minimum(a[0], v))
                    a[0] = jnp.maximum(a[0], v)
                tv = buf[row, pl.ds(984, 16)]
                tv = jnp.where(lane < 8, _NEG, tv)
                a = acc[0]
                a[1] = jnp.maximum(a[1], jnp.minimum(a[0], tv))
                a[0] = jnp.maximum(a[0], tv)
                m1v = jnp.maximum(acc[0][0], acc[1][0])
                m2v = jnp.maximum(jnp.minimum(acc[0][0], acc[1][0]),
                                  jnp.maximum(acc[0][1], acc[1][1]))
                M1 = jnp.max(m1v)
                eq = m1v == M1
                cnt = plsc.all_reduce_population_count(eq)
                r2 = jnp.maximum(jnp.max(jnp.where(eq, _NEG, m1v)),
                                 jnp.max(m2v))
                M2 = jnp.where(cnt > 1, M1, r2)
                m1p = jnp.where(lane == rr, M1, m1p)
                m2p = jnp.where(lane == rr, M2, m2p)
                return (m1p, m2p)

            m1p, m2p = lax.fori_loop(0, 16, row_body, (neg, neg))
            tgt16 = tgt_v[pl.ds(chunk * _CH + gg * 16, 16)]
            rows = gg * 16 + lane
            tlv = plsc.load_gather(buf, [rows, tgt16])
            margv = jnp.where(m1p == tlv, m1p - m2p, jnp.float32(0.0))
            marg_v[pl.ds(chunk * _CH + gg * 16, 16)] = margv
        return carry0

    def chunk_slice(g):
        return o_hbm.at[pl.ds(base + g * _CH, _CH), :]

    cp0 = pltpu.async_copy(chunk_slice(0), buf0, sem0)

    def pair_body(p, carry):
        cp1 = pltpu.async_copy(chunk_slice(2 * p + 1), buf1, sem1)
        pltpu.make_async_copy(chunk_slice(2 * p), buf0, sem0).wait()
        carry = process_chunk(buf0, 2 * p, carry)
        cp2 = pltpu.async_copy(chunk_slice(2 * p + 2), buf0, sem0)
        pltpu.make_async_copy(chunk_slice(2 * p + 1), buf1, sem1).wait()
        carry = process_chunk(buf1, 2 * p + 1, carry)
        return carry

    carry = lax.fori_loop(0, _NCHUNK // 2 - 1, pair_body, 0)
    cp1 = pltpu.async_copy(chunk_slice(_NCHUNK - 1), buf1, sem1)
    pltpu.make_async_copy(chunk_slice(_NCHUNK - 2), buf0, sem0).wait()
    carry = process_chunk(buf0, _NCHUNK - 2, carry)
    pltpu.make_async_copy(chunk_slice(_NCHUNK - 1), buf1, sem1).wait()
    carry = process_chunk(buf1, _NCHUNK - 1, carry)
    pltpu.sync_copy(marg_v, out_hbm.at[pl.ds(base, _RPW)])


@jax.jit
def _sc_margins(o_flat, targets):
    mesh = plsc.VectorSubcoreMesh(core_axis_name="c", subcore_axis_name="s")
    return pl.kernel(
        _sc_margin_body,
        mesh=mesh,
        out_type=jax.ShapeDtypeStruct((_B,), jnp.float32),
        scratch_types=[
            pltpu.VMEM((_RPW,), jnp.int32),
            pltpu.VMEM((_CH, _C), jnp.float32),
            pltpu.VMEM((_CH, _C), jnp.float32),
            pltpu.VMEM((_RPW,), jnp.float32),
            pltpu.SemaphoreType.DMA,
            pltpu.SemaphoreType.DMA,
        ],
        compiler_params=pltpu.CompilerParams(needs_layout_passes=False),
    )(o_flat, targets)


def _tc_body(o1, o2, o3, o4, tgt, out, mx):
    t = tgt[:, 0]
    col = jax.lax.broadcasted_iota(jnp.int32, (_ROWS, _C), 1)
    tmask = col == t[:, None]

    def margin(o):
        m1 = jnp.max(o, axis=1)
        tl = jnp.sum(jnp.where(tmask, o, jnp.float32(0.0)), axis=1)
        mx2 = jnp.max(jnp.where(tmask, _NEG, o), axis=1)
        return jnp.where(m1 == tl, m1 - mx2, jnp.float32(0.0)), m1

    d1, x1 = margin(o1[...])
    d2, x2 = margin(o2[...])
    d3, x3 = margin(o3[...])
    d4, x4 = margin(o4[...])
    out[...] = jnp.stack([d1, d2, d3, d4], axis=1)

    bmax = jnp.max(jnp.maximum(jnp.maximum(x1, x2), jnp.maximum(x3, x4)))

    @pl.when(pl.program_id(0) == 0)
    def _():
        mx[...] = bmax[None, None]

    @pl.when(pl.program_id(0) != 0)
    def _():
        mx[...] = jnp.maximum(mx[...], bmax[None, None])


def _combine_body(d14, d5, out):
    preds = jnp.concatenate([d14[...], d5[...]], axis=1) * jnp.float32(0.5)
    preds = preds - jnp.max(preds, axis=1, keepdims=True)
    e = jnp.exp(preds)
    out[...] = e / jnp.sum(e, axis=1, keepdims=True)


@jax.jit
def _run(o1, o2, o3, o4, o5, targets):
    d5 = _sc_margins(o5, targets)

    grid = (_B // _ROWS,)
    ospec = pl.BlockSpec((_ROWS, _C), lambda i: (i, 0))
    d14, mx = pl.pallas_call(
        _tc_body,
        grid=grid,
        in_specs=[ospec, ospec, ospec, ospec,
                  pl.BlockSpec((_ROWS, 1), lambda i: (i, 0))],
        out_specs=[pl.BlockSpec((_ROWS, 4), lambda i: (i, 0)),
                   pl.BlockSpec((1, 1), lambda i: (0, 0))],
        out_shape=[jax.ShapeDtypeStruct((_B, 4), jnp.float32),
                   jax.ShapeDtypeStruct((1, 1), jnp.float32)],
        compiler_params=pltpu.CompilerParams(
            dimension_semantics=("arbitrary",)),
    )(o1, o2, o3, o4, targets.reshape(_B, 1))

    out = pl.pallas_call(
        _combine_body,
        out_shape=jax.ShapeDtypeStruct((_B, 5), jnp.float32),
    )(d14, d5.reshape(_B, 1))
    return mx[0, 0], out


def kernel(outputs1, outputs2, outputs3, outputs4, mimic, targets, n_test):
    mx, out = _run(outputs1, outputs2, outputs3, outputs4, mimic, targets)
    return mx, out


# speedup vs baseline: 1.0161x; 1.0119x over previous
import jax
import jax.numpy as jnp
from jax import lax
from jax.experimental import pallas as pl
from jax.experimental.pallas import tpu as pltpu
from jax.experimental.pallas import tpu_sc as plsc

_B = 16384
_C = 1000
_NEG = -3.0e38

_NC = 2
_NS = 16
_NW = _NC * _NS
_RPW = _B // _NW
_CH = 32
_NCHUNK = _RPW // _CH


def _sc_margin_body(o_hbm, t_hbm, out_hbm, bm_hbm, tgt_v, buf0, buf1,
                    marg_v, bm_v, sem0, sem1):
    wid = lax.axis_index("s") * _NC + lax.axis_index("c")
    base = wid * _RPW
    pltpu.sync_copy(t_hbm.at[pl.ds(base, _RPW)], tgt_v)

    lane = lax.iota(jnp.int32, 16)
    neg = jnp.full((16,), _NEG, jnp.float32)

    def process_chunk(buf, chunk, bmv):
        for gg in range(_CH // 16):

            def row_body(rr, carry):
                m1p, m2p = carry
                row = gg * 16 + rr

                acc = [[neg, neg], [neg, neg]]
                for i in range(62):
                    v = buf[row, pl.ds(i * 16, 16)]
                    a = acc[i % 2]
                    a[1] = jnp.maximum(a[1], jnp.minimum(a[0], v))
                    a[0] = jnp.maximum(a[0], v)
                tv = buf[row, pl.ds(984, 16)]
                tv = jnp.where(lane < 8, _NEG, tv)
                a = acc[0]
                a[1] = jnp.maximum(a[1], jnp.minimum(a[0], tv))
                a[0] = jnp.maximum(a[0], tv)
                m1v = jnp.maximum(acc[0][0], acc[1][0])
                m2v = jnp.maximum(jnp.minimum(acc[0][0], acc[1][0]),
                                  jnp.maximum(acc[0][1], acc[1][1]))
                M1 = jnp.max(m1v)
                eq = m1v == M1
                cnt = plsc.all_reduce_population_count(eq)
                r2 = jnp.maximum(jnp.max(jnp.where(eq, _NEG, m1v)),
                                 jnp.max(m2v))
                M2 = jnp.where(cnt > 1, M1, r2)
                m1p = jnp.where(lane == rr, M1, m1p)
                m2p = jnp.where(lane == rr, M2, m2p)
                return (m1p, m2p)

            m1p, m2p = lax.fori_loop(0, 16, row_body, (neg, neg))
            tgt16 = tgt_v[pl.ds(chunk * _CH + gg * 16, 16)]
            rows = gg * 16 + lane
            tlv = plsc.load_gather(buf, [rows, tgt16])
            margv = jnp.where(m1p == tlv, m1p - m2p, jnp.float32(0.0))
            marg_v[pl.ds(chunk * _CH + gg * 16, 16)] = margv
            bmv = jnp.maximum(bmv, m1p)
        return bmv

    def chunk_slice(g):
        return o_hbm.at[pl.ds(base + g * _CH, _CH), :]

    pltpu.async_copy(chunk_slice(0), buf0, sem0)

    def pair_body(p, bmv):
        pltpu.async_copy(chunk_slice(2 * p + 1), buf1, sem1)
        pltpu.make_async_copy(chunk_slice(2 * p), buf0, sem0).wait()
        bmv = process_chunk(buf0, 2 * p, bmv)
        pltpu.async_copy(chunk_slice(2 * p + 2), buf0, sem0)
        pltpu.make_async_copy(chunk_slice(2 * p + 1), buf1, sem1).wait()
        bmv = process_chunk(buf1, 2 * p + 1, bmv)
        return bmv

    bmv = lax.fori_loop(0, _NCHUNK // 2 - 1, pair_body, neg)
    pltpu.async_copy(chunk_slice(_NCHUNK - 1), buf1, sem1)
    pltpu.make_async_copy(chunk_slice(_NCHUNK - 2), buf0, sem0).wait()
    bmv = process_chunk(buf0, _NCHUNK - 2, bmv)
    pltpu.make_async_copy(chunk_slice(_NCHUNK - 1), buf1, sem1).wait()
    bmv = process_chunk(buf1, _NCHUNK - 1, bmv)
    bm_v[...] = bmv
    pltpu.sync_copy(marg_v, out_hbm.at[pl.ds(base, _RPW)])
    pltpu.sync_copy(bm_v, bm_hbm.at[pl.ds(wid * 16, 16)])


def _sc_margins(o, targets):
    mesh = plsc.VectorSubcoreMesh(core_axis_name="c", subcore_axis_name="s")
    return pl.kernel(
        _sc_margin_body,
        mesh=mesh,
        out_type=[jax.ShapeDtypeStruct((_B,), jnp.float32),
                  jax.ShapeDtypeStruct((_NW * 16,), jnp.float32)],
        scratch_types=[
            pltpu.VMEM((_RPW,), jnp.int32),
            pltpu.VMEM((_CH, _C), jnp.float32),
            pltpu.VMEM((_CH, _C), jnp.float32),
            pltpu.VMEM((_RPW,), jnp.float32),
            pltpu.VMEM((16,), jnp.float32),
            pltpu.SemaphoreType.DMA,
            pltpu.SemaphoreType.DMA,
        ],
        compiler_params=pltpu.CompilerParams(needs_layout_passes=False),
    )(o, targets)


def _combine_body(d1, d2, d3, d4, d5, p1, p2, p3, p4, out, mx):
    preds = jnp.concatenate(
        [d1[...], d2[...], d3[...], d4[...], d5[...]], axis=1)
    preds = preds * jnp.float32(0.5)
    preds = preds - jnp.max(preds, axis=1, keepdims=True)
    e = jnp.exp(preds)
    out[...] = e / jnp.sum(e, axis=1, keepdims=True)

    @pl.when(pl.program_id(0) == 0)
    def _():
        pm = jnp.maximum(jnp.maximum(p1[...], p2[...]),
                         jnp.maximum(p3[...], p4[...]))
        mx[...] = jnp.max(pm)[None, None]


@jax.jit
def _run(o1, o2, o3, o4, o5, targets):
    d1, p1 = _sc_margins(o1, targets)
    d2, p2 = _sc_margins(o2, targets)
    d3, p3 = _sc_margins(o3, targets)
    d4, p4 = _sc_margins(o4, targets)
    d5, _ = _sc_margins(o5, targets)

    rows2 = 512
    dspec = pl.BlockSpec((rows2, 1), lambda i: (i, 0))
    pspec = pl.BlockSpec((_NW, 16), lambda i: (0, 0))
    out, mx = pl.pallas_call(
        _combine_body,
        grid=(_B // rows2,),
        in_specs=[dspec] * 5 + [pspec] * 4,
        out_specs=[pl.BlockSpec((rows2, 5), lambda i: (i, 0)),
                   pl.BlockSpec((1, 1), lambda i: (0, 0))],
        out_shape=[jax.ShapeDtypeStruct((_B, 5), jnp.float32),
                   jax.ShapeDtypeStruct((1, 1), jnp.float32)],
        compiler_params=pltpu.CompilerParams(
            dimension_semantics=("arbitrary",)),
    )(d1.reshape(_B, 1), d2.reshape(_B, 1), d3.reshape(_B, 1),
      d4.reshape(_B, 1), d5.reshape(_B, 1),
      p1.reshape(_NW, 16), p2.reshape(_NW, 16),
      p3.reshape(_NW, 16), p4.reshape(_NW, 16))
    return mx[0, 0], out


def kernel(outputs1, outputs2, outputs3, outputs4, mimic, targets, n_test):
    mx, out = _run(outputs1, outputs2, outputs3, outputs4, mimic, targets)
    return mx, out
